# baseline (speedup 1.0000x reference)
import functools

import jax
import jax.numpy as jnp
from jax.experimental import pallas as pl
from jax.experimental.pallas import tpu as pltpu


def _chamfer_body(x_ref, yt_ref, loss_ref, minx_ref, *, nj, inv_bn, inv_bm):
    b = pl.program_id(0)
    j = pl.program_id(1)

    x = x_ref[0]
    yt = yt_ref[0]

    xy = jax.lax.dot_general(
        x, yt, (((1,), (0,)), ((), ())),
        preferred_element_type=jnp.float32)
    x2 = jnp.sum(x * x, axis=1, keepdims=True)
    y2 = jnp.sum(yt * yt, axis=0, keepdims=True)
    d2 = (x2 + y2) - 2.0 * xy

    rowmin = jnp.min(d2, axis=1, keepdims=True)
    colmin = jnp.min(d2, axis=0, keepdims=True)

    @pl.when(j == 0)
    def _init_rows():
        minx_ref[...] = rowmin

    @pl.when(j > 0)
    def _acc_rows():
        minx_ref[...] = jnp.minimum(minx_ref[...], rowmin)

    @pl.when((b == 0) & (j == 0))
    def _init_loss():
        loss_ref[...] = jnp.zeros_like(loss_ref)

    contrib = jnp.sum(jnp.maximum(colmin, 0.0), keepdims=True) * inv_bm
    loss_ref[...] += contrib

    @pl.when(j == nj - 1)
    def _flush_rows():
        loss_ref[...] += (
            jnp.sum(jnp.maximum(minx_ref[...], 0.0), keepdims=True) * inv_bn)


def kernel(pred_points, gt_points):
    x = pred_points.astype(jnp.float32)
    y = gt_points.astype(jnp.float32)
    B, N, D = x.shape
    M = y.shape[1]
    KP = 8

    xp = jnp.concatenate(
        [x, jnp.zeros((B, N, KP - D), jnp.float32)], axis=-1)
    ytp = jnp.concatenate(
        [y, jnp.zeros((B, M, KP - D), jnp.float32)],
        axis=-1).transpose(0, 2, 1)

    MT = 512 if M % 512 == 0 else M
    nj = M // MT

    out = pl.pallas_call(
        functools.partial(
            _chamfer_body, nj=nj,
            inv_bn=1.0 / (B * N), inv_bm=1.0 / (B * M)),
        grid=(B, nj),
        in_specs=[
            pl.BlockSpec((1, N, KP), lambda b, j: (b, 0, 0)),
            pl.BlockSpec((1, KP, MT), lambda b, j: (b, 0, j)),
        ],
        out_specs=pl.BlockSpec((1, 1), lambda b, j: (0, 0)),
        out_shape=jax.ShapeDtypeStruct((1, 1), jnp.float32),
        scratch_shapes=[pltpu.VMEM((N, 1), jnp.float32)],
    )(xp, ytp)
    return out[0, 0]
